# Initial kernel scaffold; baseline (speedup 1.0000x reference)
import jax
import jax.numpy as jnp
from jax.experimental import pallas as pl


def kernel(inputs, anchors):
    raise NotImplementedError("write your pallas kernel here")


# speedup vs baseline: 3.1886x; 3.1886x over previous
import functools

import jax
import jax.numpy as jnp
from jax.experimental import pallas as pl
from jax.experimental.pallas import tpu as pltpu

_STRIDE = 16.0
_CLS_THRESH = 0.95
_MAX_IOU = 0.1
_NUM_PROPOSALS = 10
_NUM_ANCHORS = 9
_CHUNK = 2048


def _decode_body(x_ref, anchors_ref, out_ref, *, H, W, C):
    c = pl.program_id(1)
    t = x_ref[0]
    n = t.shape[0]

    a_reg = t[:, C - 10:C - 1]
    iota9 = jax.lax.broadcasted_iota(jnp.int32, (n, _NUM_ANCHORS), 1)
    m = jnp.max(a_reg, axis=1, keepdims=True)
    a_idx = jnp.min(jnp.where(a_reg == m, iota9, _NUM_ANCHORS), axis=1,
                    keepdims=True)

    breg = t[:, :C - 10]
    iota36 = jax.lax.broadcasted_iota(jnp.int32, (n, 4 * _NUM_ANCHORS), 1)
    ch0 = a_idx * 4
    d = [jnp.sum(jnp.where(iota36 == ch0 + k, breg, 0.0), axis=1,
                 keepdims=True) for k in range(4)]

    aw = jnp.zeros((n, 1), jnp.float32)
    ah = jnp.zeros((n, 1), jnp.float32)
    for j in range(_NUM_ANCHORS):
        wj = anchors_ref[j, 1]
        rj = anchors_ref[j, 0]
        sel = a_idx == j
        aw = jnp.where(sel, wj, aw)
        ah = jnp.where(sel, wj / rj, ah)

    p = c * n + jax.lax.broadcasted_iota(jnp.int32, (n, 1), 0)
    yy = p // W
    xx = p % W
    border = (yy == 0) | (yy == H - 1) | (xx == 0) | (xx == W - 1)
    score = jnp.where(border, 0.0, t[:, C - 1:C])
    ax = (xx.astype(jnp.float32) + 0.5) * _STRIDE
    ay = (yy.astype(jnp.float32) + 0.5) * _STRIDE

    cx = d[0] * aw + ax
    cy = d[1] * ah + ay
    bw = jnp.exp(d[2]) * aw
    bh = jnp.exp(d[3]) * ah
    y1 = cy - bh / 2.0
    x1 = cx - bw / 2.0
    y2 = cy + bh / 2.0
    x2 = cx + bw / 2.0

    pad = jnp.zeros((n, 16 - 9), jnp.float32)
    out_ref[0] = jnp.concatenate(
        [score, y1, x1, y2, x2, cx, cy, bw, bh, pad], axis=1)


def _nms_body(planes_ref, out_ref):
    sc = planes_ref[0, 0]
    y1 = planes_ref[0, 1]
    x1 = planes_ref[0, 2]
    y2 = planes_ref[0, 3]
    x2 = planes_ref[0, 4]
    cx = planes_ref[0, 5]
    cy = planes_ref[0, 6]
    bw = planes_ref[0, 7]
    bh = planes_ref[0, 8]
    hh, ww = sc.shape
    N = hh * ww

    lin = (jax.lax.broadcasted_iota(jnp.int32, (hh, ww), 0) * ww
           + jax.lax.broadcasted_iota(jnp.int32, (hh, ww), 1))
    area_b = jnp.maximum(0.0, y2 - y1) * jnp.maximum(0.0, x2 - x1)
    neg_inf = jnp.float32(-jnp.inf)
    masked0 = jnp.where(sc > _CLS_THRESH, sc, neg_inf)
    out0 = jnp.zeros((_NUM_PROPOSALS, 4), jnp.float32)

    rowi = jax.lax.broadcasted_iota(jnp.int32, (_NUM_PROPOSALS, 4), 0)
    coli = jax.lax.broadcasted_iota(jnp.int32, (_NUM_PROPOSALS, 4), 1)

    def body(i, carry):
        out, masked = carry
        mx = jnp.max(masked)
        any_valid = mx > 0.0
        j = jnp.min(jnp.where(masked == mx, lin, N))
        selj = lin == j

        def pick(plane):
            return jnp.sum(jnp.where(selj, plane, 0.0))

        by1 = pick(y1); bx1 = pick(x1); by2 = pick(y2); bx2 = pick(x2)
        bcx = pick(cx); bcy = pick(cy); bbw = pick(bw); bbh = pick(bh)

        iy1 = jnp.maximum(by1, y1)
        ix1 = jnp.maximum(bx1, x1)
        iy2 = jnp.minimum(by2, y2)
        ix2 = jnp.minimum(bx2, x2)
        inter = jnp.maximum(0.0, iy2 - iy1) * jnp.maximum(0.0, ix2 - ix1)
        area_a = jnp.maximum(0.0, by2 - by1) * jnp.maximum(0.0, bx2 - bx1)
        union = area_a + area_b - inter
        iou = jnp.where(union > 0.0, inter / union, 0.0)

        new_masked = jnp.where((iou > _MAX_IOU) | (lin == j), neg_inf, masked)
        masked = jnp.where(any_valid, new_masked, masked)
        vals = jnp.where(coli == 0, bcx,
                         jnp.where(coli == 1, bcy,
                                   jnp.where(coli == 2, bbw, bbh)))
        out = jnp.where((rowi == i) & any_valid, vals, out)
        return out, masked

    out, _ = jax.lax.fori_loop(0, _NUM_PROPOSALS, body, (out0, masked0))
    out_ref[0] = out


@jax.jit
def kernel(inputs, anchors):
    B, H, W, C = inputs.shape
    HW = H * W
    x = inputs.reshape(B, HW, C)
    n_chunks = HW // _CHUNK

    decoded = pl.pallas_call(
        functools.partial(_decode_body, H=H, W=W, C=C),
        grid=(B, n_chunks),
        in_specs=[
            pl.BlockSpec((1, _CHUNK, C), lambda b, c: (b, c, 0)),
            pl.BlockSpec(memory_space=pltpu.SMEM),
        ],
        out_specs=pl.BlockSpec((1, _CHUNK, 16), lambda b, c: (b, c, 0)),
        out_shape=jax.ShapeDtypeStruct((B, HW, 16), jnp.float32),
    )(x, anchors)

    planes = decoded.transpose(0, 2, 1).reshape(B, 16, H, W)

    out = pl.pallas_call(
        _nms_body,
        grid=(B,),
        in_specs=[pl.BlockSpec((1, 16, H, W), lambda b: (b, 0, 0, 0))],
        out_specs=pl.BlockSpec((1, _NUM_PROPOSALS, 4), lambda b: (b, 0, 0)),
        out_shape=jax.ShapeDtypeStruct((B, _NUM_PROPOSALS, 4), jnp.float32),
    )(planes)
    return out


# speedup vs baseline: 10.4168x; 3.2668x over previous
import functools

import jax
import jax.numpy as jnp
from jax.experimental import pallas as pl
from jax.experimental.pallas import tpu as pltpu

_STRIDE = 16.0
_CLS_THRESH = 0.95
_MAX_IOU = 0.1
_NUM_PROPOSALS = 10
_NUM_ANCHORS = 9
_CHUNK = 2048


def _decode_body(x_ref, anchors_ref, out_ref, *, H, W, C):
    c = pl.program_id(1)
    t = x_ref[0]
    n = t.shape[0]
    tT = t.T

    a_reg = tT[C - 10:C - 1]
    iota9 = jax.lax.broadcasted_iota(jnp.int32, (_NUM_ANCHORS, n), 0)
    m = jnp.max(a_reg, axis=0, keepdims=True)
    a_idx = jnp.min(jnp.where(a_reg == m, iota9, _NUM_ANCHORS), axis=0,
                    keepdims=True)
    selj = [a_idx == j for j in range(_NUM_ANCHORS)]
    d = []
    for k in range(4):
        acc = jnp.zeros((1, n), jnp.float32)
        for j in range(_NUM_ANCHORS):
            acc = jnp.where(selj[j], tT[4 * j + k:4 * j + k + 1], acc)
        d.append(acc)

    aw = jnp.zeros((1, n), jnp.float32)
    ah = jnp.zeros((1, n), jnp.float32)
    for j in range(_NUM_ANCHORS):
        wj = anchors_ref[j, 1]
        rj = anchors_ref[j, 0]
        sj = a_idx == j
        aw = jnp.where(sj, wj, aw)
        ah = jnp.where(sj, wj / rj, ah)

    p = c * n + jax.lax.broadcasted_iota(jnp.int32, (1, n), 1)
    yy = p // W
    xx = p % W
    border = (yy == 0) | (yy == H - 1) | (xx == 0) | (xx == W - 1)
    score = jnp.where(border, 0.0, tT[C - 1:C])
    ax = (xx.astype(jnp.float32) + 0.5) * _STRIDE
    ay = (yy.astype(jnp.float32) + 0.5) * _STRIDE

    cx = d[0] * aw + ax
    cy = d[1] * ah + ay
    bw = jnp.exp(d[2]) * aw
    bh = jnp.exp(d[3]) * ah
    y1 = cy - bh / 2.0
    x1 = cx - bw / 2.0
    y2 = cy + bh / 2.0
    x2 = cx + bw / 2.0

    pad = jnp.zeros((16 - 9, n), jnp.float32)
    out_ref[0] = jnp.concatenate(
        [score, y1, x1, y2, x2, cx, cy, bw, bh, pad], axis=0)


def _nms_body(planes_ref, out_ref):
    sc = planes_ref[0, 0]
    y1 = planes_ref[0, 1]
    x1 = planes_ref[0, 2]
    y2 = planes_ref[0, 3]
    x2 = planes_ref[0, 4]
    cx = planes_ref[0, 5]
    cy = planes_ref[0, 6]
    bw = planes_ref[0, 7]
    bh = planes_ref[0, 8]
    hh, ww = sc.shape
    N = hh * ww

    lin = (jax.lax.broadcasted_iota(jnp.int32, (hh, ww), 0) * ww
           + jax.lax.broadcasted_iota(jnp.int32, (hh, ww), 1))
    area_b = jnp.maximum(0.0, y2 - y1) * jnp.maximum(0.0, x2 - x1)
    neg_inf = jnp.float32(-jnp.inf)
    masked0 = jnp.where(sc > _CLS_THRESH, sc, neg_inf)
    out0 = jnp.zeros((_NUM_PROPOSALS, 4), jnp.float32)

    rowi = jax.lax.broadcasted_iota(jnp.int32, (_NUM_PROPOSALS, 4), 0)
    coli = jax.lax.broadcasted_iota(jnp.int32, (_NUM_PROPOSALS, 4), 1)

    def body(i, carry):
        out, masked = carry
        mx = jnp.max(masked)
        any_valid = mx > 0.0
        j = jnp.min(jnp.where(masked == mx, lin, N))
        selj = lin == j

        def pick(plane):
            return jnp.sum(jnp.where(selj, plane, 0.0))

        by1 = pick(y1); bx1 = pick(x1); by2 = pick(y2); bx2 = pick(x2)
        bcx = pick(cx); bcy = pick(cy); bbw = pick(bw); bbh = pick(bh)

        iy1 = jnp.maximum(by1, y1)
        ix1 = jnp.maximum(bx1, x1)
        iy2 = jnp.minimum(by2, y2)
        ix2 = jnp.minimum(bx2, x2)
        inter = jnp.maximum(0.0, iy2 - iy1) * jnp.maximum(0.0, ix2 - ix1)
        area_a = jnp.maximum(0.0, by2 - by1) * jnp.maximum(0.0, bx2 - bx1)
        union = area_a + area_b - inter
        iou = jnp.where(union > 0.0, inter / union, 0.0)

        new_masked = jnp.where((iou > _MAX_IOU) | (lin == j), neg_inf, masked)
        masked = jnp.where(any_valid, new_masked, masked)
        vals = jnp.where(coli == 0, bcx,
                         jnp.where(coli == 1, bcy,
                                   jnp.where(coli == 2, bbw, bbh)))
        out = jnp.where((rowi == i) & any_valid, vals, out)
        return out, masked

    out, _ = jax.lax.fori_loop(0, _NUM_PROPOSALS, body, (out0, masked0))
    out_ref[0] = out


@jax.jit
def kernel(inputs, anchors):
    B, H, W, C = inputs.shape
    HW = H * W
    x = inputs.reshape(B, HW, C)
    n_chunks = HW // _CHUNK

    decoded = pl.pallas_call(
        functools.partial(_decode_body, H=H, W=W, C=C),
        grid=(B, n_chunks),
        in_specs=[
            pl.BlockSpec((1, _CHUNK, C), lambda b, c: (b, c, 0)),
            pl.BlockSpec(memory_space=pltpu.SMEM),
        ],
        out_specs=pl.BlockSpec((1, 16, _CHUNK), lambda b, c: (b, 0, c)),
        out_shape=jax.ShapeDtypeStruct((B, 16, HW), jnp.float32),
    )(x, anchors)

    planes = decoded.reshape(B, 16, H, W)

    out = pl.pallas_call(
        _nms_body,
        grid=(B,),
        in_specs=[pl.BlockSpec((1, 16, H, W), lambda b: (b, 0, 0, 0))],
        out_specs=pl.BlockSpec((1, _NUM_PROPOSALS, 4), lambda b: (b, 0, 0)),
        out_shape=jax.ShapeDtypeStruct((B, _NUM_PROPOSALS, 4), jnp.float32),
    )(planes)
    return out


# speedup vs baseline: 32.5313x; 3.1230x over previous
import functools

import jax
import jax.numpy as jnp
from jax.experimental import pallas as pl
from jax.experimental.pallas import tpu as pltpu

_STRIDE = 16.0
_CLS_THRESH = 0.95
_MAX_IOU = 0.1
_NUM_PROPOSALS = 10
_NUM_ANCHORS = 9


def _body(x_ref, anchors_ref, out_ref, *, H, W, C):
    t = x_ref[0]

    a = [t[C - 10 + j] for j in range(_NUM_ANCHORS)]
    m = a[0]
    for j in range(1, _NUM_ANCHORS):
        m = jnp.maximum(m, a[j])
    a_idx = jnp.full((H, W), _NUM_ANCHORS, jnp.int32)
    for j in range(_NUM_ANCHORS - 1, -1, -1):
        a_idx = jnp.where(a[j] == m, j, a_idx)

    selj = [a_idx == j for j in range(_NUM_ANCHORS)]
    d = []
    for k in range(4):
        acc = t[k]
        for j in range(1, _NUM_ANCHORS):
            acc = jnp.where(selj[j], t[4 * j + k], acc)
        d.append(acc)
    aw = jnp.zeros((H, W), jnp.float32)
    ah = jnp.zeros((H, W), jnp.float32)
    for j in range(_NUM_ANCHORS):
        wj = anchors_ref[j, 1]
        rj = anchors_ref[j, 0]
        aw = jnp.where(selj[j], wj, aw)
        ah = jnp.where(selj[j], wj / rj, ah)

    yi = jax.lax.broadcasted_iota(jnp.int32, (H, W), 0)
    xi = jax.lax.broadcasted_iota(jnp.int32, (H, W), 1)
    border = (yi == 0) | (yi == H - 1) | (xi == 0) | (xi == W - 1)
    sc = jnp.where(border, 0.0, t[C - 1])
    ax = (xi.astype(jnp.float32) + 0.5) * _STRIDE
    ay = (yi.astype(jnp.float32) + 0.5) * _STRIDE

    cx = d[0] * aw + ax
    cy = d[1] * ah + ay
    bw = jnp.exp(d[2]) * aw
    bh = jnp.exp(d[3]) * ah
    y1 = cy - bh / 2.0
    x1 = cx - bw / 2.0
    y2 = cy + bh / 2.0
    x2 = cx + bw / 2.0

    lin = yi * W + xi
    N = H * W
    area_b = jnp.maximum(0.0, y2 - y1) * jnp.maximum(0.0, x2 - x1)
    neg_inf = jnp.float32(-jnp.inf)
    masked0 = jnp.where(sc > _CLS_THRESH, sc, neg_inf)
    out0 = jnp.zeros((_NUM_PROPOSALS, 4), jnp.float32)

    rowi = jax.lax.broadcasted_iota(jnp.int32, (_NUM_PROPOSALS, 4), 0)
    coli = jax.lax.broadcasted_iota(jnp.int32, (_NUM_PROPOSALS, 4), 1)

    def body(i, carry):
        out, masked = carry
        mx = jnp.max(masked)
        any_valid = mx > 0.0
        j = jnp.min(jnp.where(masked == mx, lin, N))
        selp = lin == j

        def pick(plane):
            return jnp.sum(jnp.where(selp, plane, 0.0))

        by1 = pick(y1); bx1 = pick(x1); by2 = pick(y2); bx2 = pick(x2)
        bcx = pick(cx); bcy = pick(cy); bbw = pick(bw); bbh = pick(bh)

        iy1 = jnp.maximum(by1, y1)
        ix1 = jnp.maximum(bx1, x1)
        iy2 = jnp.minimum(by2, y2)
        ix2 = jnp.minimum(bx2, x2)
        inter = jnp.maximum(0.0, iy2 - iy1) * jnp.maximum(0.0, ix2 - ix1)
        area_a = jnp.maximum(0.0, by2 - by1) * jnp.maximum(0.0, bx2 - bx1)
        union = area_a + area_b - inter
        iou = jnp.where(union > 0.0, inter / union, 0.0)

        new_masked = jnp.where((iou > _MAX_IOU) | (lin == j), neg_inf, masked)
        masked = jnp.where(any_valid, new_masked, masked)
        vals = jnp.where(coli == 0, bcx,
                         jnp.where(coli == 1, bcy,
                                   jnp.where(coli == 2, bbw, bbh)))
        out = jnp.where((rowi == i) & any_valid, vals, out)
        return out, masked

    out, _ = jax.lax.fori_loop(0, _NUM_PROPOSALS, body, (out0, masked0))
    out_ref[0] = out


@jax.jit
def kernel(inputs, anchors):
    B, H, W, C = inputs.shape
    xp = inputs.transpose(0, 3, 1, 2)

    return pl.pallas_call(
        functools.partial(_body, H=H, W=W, C=C),
        grid=(B,),
        in_specs=[
            pl.BlockSpec((1, C, H, W), lambda b: (b, 0, 0, 0)),
            pl.BlockSpec(memory_space=pltpu.SMEM),
        ],
        out_specs=pl.BlockSpec((1, _NUM_PROPOSALS, 4), lambda b: (b, 0, 0)),
        out_shape=jax.ShapeDtypeStruct((B, _NUM_PROPOSALS, 4), jnp.float32),
    )(xp, anchors)


# speedup vs baseline: 49.7727x; 1.5300x over previous
import functools

import jax
import jax.numpy as jnp
from jax.experimental import pallas as pl
from jax.experimental.pallas import tpu as pltpu

_STRIDE = 16.0
_CLS_THRESH = 0.95
_MAX_IOU = 0.1
_NUM_PROPOSALS = 10
_NUM_ANCHORS = 9


def _body(x_ref, anchors_ref, out_ref, scr_ref, *, B, H, W, C):
    b = pl.program_id(0)
    t = x_ref[0]

    a = [t[C - 10 + j] for j in range(_NUM_ANCHORS)]
    m = a[0]
    for j in range(1, _NUM_ANCHORS):
        m = jnp.maximum(m, a[j])
    a_idx = jnp.full((H, W), _NUM_ANCHORS, jnp.int32)
    for j in range(_NUM_ANCHORS - 1, -1, -1):
        a_idx = jnp.where(a[j] == m, j, a_idx)

    selj = [a_idx == j for j in range(_NUM_ANCHORS)]
    d = []
    for k in range(4):
        acc = t[k]
        for j in range(1, _NUM_ANCHORS):
            acc = jnp.where(selj[j], t[4 * j + k], acc)
        d.append(acc)
    aw = jnp.zeros((H, W), jnp.float32)
    ah = jnp.zeros((H, W), jnp.float32)
    for j in range(_NUM_ANCHORS):
        wj = anchors_ref[j, 1]
        rj = anchors_ref[j, 0]
        aw = jnp.where(selj[j], wj, aw)
        ah = jnp.where(selj[j], wj / rj, ah)

    yi = jax.lax.broadcasted_iota(jnp.int32, (H, W), 0)
    xi = jax.lax.broadcasted_iota(jnp.int32, (H, W), 1)
    border = (yi == 0) | (yi == H - 1) | (xi == 0) | (xi == W - 1)
    sc = jnp.where(border, 0.0, t[C - 1])
    ax = (xi.astype(jnp.float32) + 0.5) * _STRIDE
    ay = (yi.astype(jnp.float32) + 0.5) * _STRIDE

    cx = d[0] * aw + ax
    cy = d[1] * ah + ay
    bw = jnp.exp(d[2]) * aw
    bh = jnp.exp(d[3]) * ah
    y1 = cy - bh / 2.0
    x1 = cx - bw / 2.0
    y2 = cy + bh / 2.0
    x2 = cx + bw / 2.0

    neg_inf = jnp.float32(-jnp.inf)
    planes = [jnp.where(sc > _CLS_THRESH, sc, neg_inf),
              y1, x1, y2, x2, cx, cy, bw, bh]
    for k in range(9):
        scr_ref[k, b] = planes[k]

    @pl.when(b == B - 1)
    def _nms():
        msk0 = scr_ref[0]
        py1 = scr_ref[1]
        px1 = scr_ref[2]
        py2 = scr_ref[3]
        px2 = scr_ref[4]
        pcx = scr_ref[5]
        pcy = scr_ref[6]
        pbw = scr_ref[7]
        pbh = scr_ref[8]
        N = H * W
        lin = (yi * W + xi)[None]
        area_b = jnp.maximum(0.0, py2 - py1) * jnp.maximum(0.0, px2 - px1)
        out0 = jnp.zeros((B, _NUM_PROPOSALS, 4), jnp.float32)
        rowi = jax.lax.broadcasted_iota(jnp.int32, (B, _NUM_PROPOSALS, 4), 1)
        coli = jax.lax.broadcasted_iota(jnp.int32, (B, _NUM_PROPOSALS, 4), 2)

        def red(x, op):
            r = op(x, axis=2, keepdims=True)
            return op(r, axis=1, keepdims=True)

        def body2(i, carry):
            out, masked = carry
            mx = red(masked, jnp.max)
            any_valid = mx > 0.0
            j = red(jnp.where(masked == mx, lin, N), jnp.min)
            selp = lin == j

            def pick(plane):
                return red(jnp.where(selp, plane, 0.0), jnp.sum)

            by1 = pick(py1); bx1 = pick(px1); by2 = pick(py2); bx2 = pick(px2)
            bcx = pick(pcx); bcy = pick(pcy); bbw = pick(pbw); bbh = pick(pbh)

            iy1 = jnp.maximum(by1, py1)
            ix1 = jnp.maximum(bx1, px1)
            iy2 = jnp.minimum(by2, py2)
            ix2 = jnp.minimum(bx2, px2)
            inter = (jnp.maximum(0.0, iy2 - iy1)
                     * jnp.maximum(0.0, ix2 - ix1))
            area_a = (jnp.maximum(0.0, by2 - by1)
                      * jnp.maximum(0.0, bx2 - bx1))
            union = area_a + area_b - inter
            iou = jnp.where(union > 0.0, inter / union, 0.0)

            new_masked = jnp.where((iou > _MAX_IOU) | (lin == j),
                                   neg_inf, masked)
            masked = jnp.where(any_valid, new_masked, masked)

            vals = jnp.where(coli == 0, bcx, jnp.where(coli == 1, bcy,
                             jnp.where(coli == 2, bbw, bbh)))
            upd = (rowi == i) & any_valid
            out = jnp.where(upd, vals, out)
            return out, masked

        out, _ = jax.lax.fori_loop(0, _NUM_PROPOSALS, body2, (out0, msk0))
        out_ref[...] = out


@jax.jit
def kernel(inputs, anchors):
    B, H, W, C = inputs.shape
    xp = inputs.transpose(0, 3, 1, 2)

    return pl.pallas_call(
        functools.partial(_body, B=B, H=H, W=W, C=C),
        grid=(B,),
        in_specs=[
            pl.BlockSpec((1, C, H, W), lambda b: (b, 0, 0, 0)),
            pl.BlockSpec(memory_space=pltpu.SMEM),
        ],
        out_specs=pl.BlockSpec((B, _NUM_PROPOSALS, 4), lambda b: (0, 0, 0)),
        out_shape=jax.ShapeDtypeStruct((B, _NUM_PROPOSALS, 4), jnp.float32),
        scratch_shapes=[pltpu.VMEM((9, B, H, W), jnp.float32)],
    )(xp, anchors)


# speedup vs baseline: 59.1016x; 1.1874x over previous
import functools

import jax
import jax.numpy as jnp
from jax.experimental import pallas as pl
from jax.experimental.pallas import tpu as pltpu

_STRIDE = 16.0
_CLS_THRESH = 0.95
_MAX_IOU = 0.1
_NUM_PROPOSALS = 10
_NUM_ANCHORS = 9


def _body(x_ref, anchors_ref, out_ref, scr_ref, *, B, H, W, C):
    b = pl.program_id(0)
    t = x_ref[0]

    a = [t[C - 10 + j] for j in range(_NUM_ANCHORS)]
    m = a[0]
    for j in range(1, _NUM_ANCHORS):
        m = jnp.maximum(m, a[j])
    a_idx = jnp.full((H, W), _NUM_ANCHORS, jnp.int32)
    for j in range(_NUM_ANCHORS - 1, -1, -1):
        a_idx = jnp.where(a[j] == m, j, a_idx)

    selj = [a_idx == j for j in range(_NUM_ANCHORS)]
    d = []
    for k in range(4):
        acc = t[k]
        for j in range(1, _NUM_ANCHORS):
            acc = jnp.where(selj[j], t[4 * j + k], acc)
        d.append(acc)
    aw = jnp.zeros((H, W), jnp.float32)
    ah = jnp.zeros((H, W), jnp.float32)
    for j in range(_NUM_ANCHORS):
        wj = anchors_ref[j, 1]
        rj = anchors_ref[j, 0]
        aw = jnp.where(selj[j], wj, aw)
        ah = jnp.where(selj[j], wj / rj, ah)

    yi = jax.lax.broadcasted_iota(jnp.int32, (H, W), 0)
    xi = jax.lax.broadcasted_iota(jnp.int32, (H, W), 1)
    border = (yi == 0) | (yi == H - 1) | (xi == 0) | (xi == W - 1)
    sc = jnp.where(border, 0.0, t[C - 1])
    ax = (xi.astype(jnp.float32) + 0.5) * _STRIDE
    ay = (yi.astype(jnp.float32) + 0.5) * _STRIDE

    cx = d[0] * aw + ax
    cy = d[1] * ah + ay
    bw = jnp.exp(d[2]) * aw
    bh = jnp.exp(d[3]) * ah
    y1 = cy - bh / 2.0
    x1 = cx - bw / 2.0
    y2 = cy + bh / 2.0
    x2 = cx + bw / 2.0

    neg_inf = jnp.float32(-jnp.inf)
    areab = jnp.maximum(0.0, y2 - y1) * jnp.maximum(0.0, x2 - x1)
    planes = [jnp.where(sc > _CLS_THRESH, sc, neg_inf),
              y1, x1, y2, x2, cx, cy, bw, bh, areab]
    for k in range(10):
        scr_ref[k, b] = planes[k]

    @pl.when(b == B - 1)
    def _nms():
        msk0 = scr_ref[0]
        py1 = scr_ref[1]
        px1 = scr_ref[2]
        py2 = scr_ref[3]
        px2 = scr_ref[4]
        pcx = scr_ref[5]
        pcy = scr_ref[6]
        pbw = scr_ref[7]
        pbh = scr_ref[8]
        area_b = scr_ref[9]
        N = H * W
        lin = (yi * W + xi)[None]
        out0 = jnp.zeros((B, _NUM_PROPOSALS, 4), jnp.float32)
        rowi = jax.lax.broadcasted_iota(jnp.int32, (B, _NUM_PROPOSALS, 4), 1)
        coli = jax.lax.broadcasted_iota(jnp.int32, (B, _NUM_PROPOSALS, 4), 2)

        def red(x, op):
            r = op(x, axis=2, keepdims=True)
            return op(r, axis=1, keepdims=True)

        def body2(i, carry):
            out, masked = carry
            mx = red(masked, jnp.max)
            any_valid = mx > 0.0
            j = red(jnp.where(masked == mx, lin, N), jnp.min)
            selp = lin == j

            def pick(plane):
                return red(jnp.where(selp, plane, 0.0), jnp.sum)

            bcx = pick(pcx); bcy = pick(pcy); bbw = pick(pbw); bbh = pick(pbh)
            by1 = bcy - bbh / 2.0
            bx1 = bcx - bbw / 2.0
            by2 = bcy + bbh / 2.0
            bx2 = bcx + bbw / 2.0

            iy1 = jnp.maximum(by1, py1)
            ix1 = jnp.maximum(bx1, px1)
            iy2 = jnp.minimum(by2, py2)
            ix2 = jnp.minimum(bx2, px2)
            inter = (jnp.maximum(0.0, iy2 - iy1)
                     * jnp.maximum(0.0, ix2 - ix1))
            area_a = (jnp.maximum(0.0, by2 - by1)
                      * jnp.maximum(0.0, bx2 - bx1))
            union = area_a + area_b - inter
            iou = jnp.where(union > 0.0, inter / union, 0.0)

            supp = ((iou > _MAX_IOU) | (lin == j)) & any_valid
            masked = jnp.where(supp, neg_inf, masked)

            vals = jnp.where(coli == 0, bcx, jnp.where(coli == 1, bcy,
                             jnp.where(coli == 2, bbw, bbh)))
            upd = (rowi == i) & any_valid
            out = jnp.where(upd, vals, out)
            return out, masked

        out, _ = jax.lax.fori_loop(0, _NUM_PROPOSALS, body2, (out0, msk0))
        out_ref[...] = out


@jax.jit
def kernel(inputs, anchors):
    B, H, W, C = inputs.shape
    xp = inputs.transpose(0, 3, 1, 2)

    return pl.pallas_call(
        functools.partial(_body, B=B, H=H, W=W, C=C),
        grid=(B,),
        in_specs=[
            pl.BlockSpec((1, C, H, W), lambda b: (b, 0, 0, 0)),
            pl.BlockSpec(memory_space=pltpu.SMEM),
        ],
        out_specs=pl.BlockSpec((B, _NUM_PROPOSALS, 4), lambda b: (0, 0, 0)),
        out_shape=jax.ShapeDtypeStruct((B, _NUM_PROPOSALS, 4), jnp.float32),
        scratch_shapes=[pltpu.VMEM((10, B, H, W), jnp.float32)],
    )(xp, anchors)
